# baseline (speedup 1.0000x reference)
import functools

import jax
import jax.numpy as jnp
from jax import lax
from jax.experimental import pallas as pl
from jax.experimental.pallas import tpu as pltpu
from jax.experimental.pallas import tpu_sc as plsc

NUM_CORES = 2
NUM_SUBCORES = 16
NW = NUM_CORES * NUM_SUBCORES
CHUNK = 128
NBUF = 4
BLK = 512


def _tc_repack(table_t):
    d, v = table_t.shape

    def body(in_ref, out_ref):
        x = in_ref[...]
        eye = jnp.eye(d, dtype=jnp.float32)
        xt = jax.lax.dot_general(
            x, eye, (((0,), (0,)), ((), ())), preferred_element_type=jnp.float32
        )
        xt3 = xt.reshape(BLK // 2, 2, d)
        out_ref[...] = jnp.concatenate([xt3[:, 0, :], xt3[:, 1, :]], axis=1)

    return pl.pallas_call(
        body,
        grid=(pl.cdiv(v, BLK),),
        in_specs=[pl.BlockSpec((d, BLK), lambda i: (0, i))],
        out_specs=pl.BlockSpec((BLK // 2, 2 * d), lambda i: (i, 0)),
        out_shape=jax.ShapeDtypeStruct((v // 2, 2 * d), jnp.float32),
    )(table_t)


def _build(B, L, D, V):
    rows_per_w = B // NW
    rest = L - CHUNK
    mesh = plsc.VectorSubcoreMesh(core_axis_name="c", subcore_axis_name="s")

    @functools.partial(
        pl.kernel,
        mesh=mesh,
        out_type=jax.ShapeDtypeStruct((B, L, 2 * D), jnp.float32),
        scratch_types=[
            pltpu.VMEM((rows_per_w, CHUNK), jnp.int32),
            pltpu.VMEM((rows_per_w, rest), jnp.int32),
            pltpu.VMEM((NBUF, L, D), jnp.float32),
            [pltpu.SemaphoreType.DMA] * NBUF,
            [pltpu.SemaphoreType.DMA] * NBUF,
        ],
        compiler_params=pltpu.CompilerParams(use_tc_tiling_on_sc=False),
    )
    def k(y_hbm, table_hbm, out_hbm, idx_a, idx_b, rows_v, gsems, osems):
        wid = lax.axis_index("s") * NUM_CORES + lax.axis_index("c")
        base = wid * rows_per_w
        pltpu.sync_copy(y_hbm.at[pl.ds(base, rows_per_w), pl.ds(0, CHUNK)], idx_a)
        pltpu.sync_copy(y_hbm.at[pl.ds(base, rows_per_w), pl.ds(CHUNK, rest)], idx_b)

        def fire(i, d):
            pltpu.async_copy(
                table_hbm.at[idx_a.at[i]], rows_v.at[d, pl.ds(0, CHUNK)], gsems[d]
            )
            pltpu.async_copy(
                table_hbm.at[idx_b.at[i]], rows_v.at[d, pl.ds(CHUNK, rest)], gsems[d]
            )

        def drain_gathers(i, d):
            pltpu.make_async_copy(
                out_hbm.at[base + i, pl.ds(0, L), pl.ds(0, D)], rows_v.at[d], gsems[d]
            ).wait()

        for d in range(NBUF):
            fire(d, d)

        def body(g, carry):
            for d in range(NBUF):
                i = g * NBUF + d
                drain_gathers(i, d)
                pltpu.async_copy(
                    rows_v.at[d], out_hbm.at[base + i, pl.ds(0, L), pl.ds(0, D)],
                    osems[d],
                )

                @pl.when(i + NBUF < rows_per_w)
                def _():
                    pltpu.make_async_copy(
                        rows_v.at[d], out_hbm.at[base + i, pl.ds(0, L), pl.ds(0, D)],
                        osems[d],
                    ).wait()
                    fire(i + NBUF, d)

            return carry

        lax.fori_loop(0, rows_per_w // NBUF, body, 0)
        for d in range(NBUF):
            i = rows_per_w - NBUF + d
            pltpu.make_async_copy(
                rows_v.at[d], out_hbm.at[base + i, pl.ds(0, L), pl.ds(0, D)],
                osems[d],
            ).wait()

    return k


@jax.jit
def kernel(y, table):
    B, L = y.shape
    V, D = table.shape
    table_lin = _tc_repack(table.T).reshape(V, D)
    return _build(B, L, D, V)(y, table_lin)[:, :, :D]


# speedup vs baseline: 1.8097x; 1.8097x over previous
import functools

import jax
import jax.numpy as jnp
from jax import lax
from jax.experimental import pallas as pl
from jax.experimental.pallas import tpu as pltpu
from jax.experimental.pallas import tpu_sc as plsc

NUM_CORES = 2
NUM_SUBCORES = 16
NW = NUM_CORES * NUM_SUBCORES
CHUNK = 128
NBUF = 4


def _build(B, L, D, V):
    rows_per_w = B // NW
    rest = L - CHUNK
    mesh = plsc.VectorSubcoreMesh(core_axis_name="c", subcore_axis_name="s")

    @functools.partial(
        pl.kernel,
        mesh=mesh,
        out_type=jax.ShapeDtypeStruct((B, L, 2 * D), jnp.float32),
        scratch_types=[
            pltpu.VMEM((rows_per_w, CHUNK), jnp.int32),
            pltpu.VMEM((rows_per_w, rest), jnp.int32),
            pltpu.VMEM((NBUF, L, D), jnp.float32),
            [pltpu.SemaphoreType.DMA] * NBUF,
            [pltpu.SemaphoreType.DMA] * NBUF,
        ],
        compiler_params=pltpu.CompilerParams(use_tc_tiling_on_sc=False),
    )
    def k(y_hbm, table_hbm, out_hbm, idx_a, idx_b, rows_v, gsems, osems):
        wid = lax.axis_index("s") * NUM_CORES + lax.axis_index("c")
        base = wid * rows_per_w
        pltpu.sync_copy(y_hbm.at[pl.ds(base, rows_per_w), pl.ds(0, CHUNK)], idx_a)
        pltpu.sync_copy(y_hbm.at[pl.ds(base, rows_per_w), pl.ds(CHUNK, rest)], idx_b)

        def fire(i, d):
            pltpu.async_copy(
                table_hbm.at[idx_a.at[i]], rows_v.at[d, pl.ds(0, CHUNK)], gsems[d]
            )
            pltpu.async_copy(
                table_hbm.at[idx_b.at[i]], rows_v.at[d, pl.ds(CHUNK, rest)], gsems[d]
            )

        def drain_gathers(i, d):
            pltpu.make_async_copy(
                out_hbm.at[base + i, pl.ds(0, L), pl.ds(0, D)], rows_v.at[d], gsems[d]
            ).wait()

        for d in range(NBUF):
            fire(d, d)

        def body(g, carry):
            for d in range(NBUF):
                i = g * NBUF + d
                drain_gathers(i, d)
                pltpu.async_copy(
                    rows_v.at[d], out_hbm.at[base + i, pl.ds(0, L), pl.ds(0, D)],
                    osems[d],
                )

                @pl.when(i + NBUF < rows_per_w)
                def _():
                    pltpu.make_async_copy(
                        rows_v.at[d], out_hbm.at[base + i, pl.ds(0, L), pl.ds(0, D)],
                        osems[d],
                    ).wait()
                    fire(i + NBUF, d)

            return carry

        lax.fori_loop(0, rows_per_w // NBUF, body, 0)
        for d in range(NBUF):
            i = rows_per_w - NBUF + d
            pltpu.make_async_copy(
                rows_v.at[d], out_hbm.at[base + i, pl.ds(0, L), pl.ds(0, D)],
                osems[d],
            ).wait()

    return k


@jax.jit
def kernel(y, table):
    B, L = y.shape
    V, D = table.shape
    return _build(B, L, D, V)(y, table)[:, :, :D]
